# baseline (speedup 1.0000x reference)
import functools

import jax
import jax.numpy as jnp
from jax import lax
from jax.experimental import pallas as pl
from jax.experimental.pallas import tpu as pltpu
from jax.experimental.pallas import tpu_sc as plsc

_F = 26
_EMB = 32
_VOCAB = 100000
_NN = 13
_VBLK = 512
_NBLK = (_VOCAB + _VBLK - 1) // _VBLK
_RPF = _NBLK * (_VBLK // 4)
_S = 16
_C = _F * _S


def _proj_body(t_ref, w_ref, out_ref):
    t = t_ref[0].astype(jnp.bfloat16)
    w = w_ref[0].astype(jnp.bfloat16)
    p = lax.dot_general(t, w, (((0,), (0,)), ((), ())),
                        preferred_element_type=jnp.float32)
    out_ref[0, :, 0:32] = p[0:128]
    out_ref[0, :, 32:64] = p[128:256]
    out_ref[0, :, 64:96] = p[256:384]
    out_ref[0, :, 96:128] = p[384:512]


@jax.jit
def _tc_project(tables_t, w1e):
    return pl.pallas_call(
        _proj_body,
        grid=(_F, _NBLK),
        in_specs=[
            pl.BlockSpec((1, _EMB, _VBLK), lambda f, v: (f, 0, v)),
            pl.BlockSpec((1, _EMB, 32), lambda f, v: (f, 0, 0)),
        ],
        out_specs=pl.BlockSpec((1, _VBLK // 4, 128), lambda f, v: (f, v, 0)),
        out_shape=jax.ShapeDtypeStruct((_F, _RPF, 128), jnp.float32),
        compiler_params=pltpu.CompilerParams(
            dimension_semantics=("arbitrary", "arbitrary"),
            fuse_transposed_lhs_in_matmul=True),
    )(tables_t, w1e)


@functools.partial(jax.jit, static_argnums=(3,))
def _sc_gather_acc(packflat, prow, sub32, batch):
    info = plsc.get_sparse_core_info()
    nc = info.num_cores
    nw = nc * info.num_subcores
    per_w = batch // nw
    n_chunks = per_w // _S
    lk_w = per_w * _F

    mesh = plsc.VectorSubcoreMesh(core_axis_name="c", subcore_axis_name="s")

    @functools.partial(
        pl.kernel,
        mesh=mesh,
        out_type=jax.ShapeDtypeStruct((nw, _EMB, per_w), jnp.float32),
        scratch_types=[
            pltpu.VMEM((_C,), jnp.int32),
            pltpu.VMEM((_C,), jnp.int32),
            pltpu.VMEM((_C,), jnp.int32),
            pltpu.VMEM((_C,), jnp.int32),
            pltpu.VMEM((_C, 128), jnp.float32),
            pltpu.VMEM((_C, 128), jnp.float32),
            pltpu.VMEM((_EMB, per_w), jnp.float32),
            pltpu.SemaphoreType.DMA,
            pltpu.SemaphoreType.DMA,
        ],
        compiler_params=pltpu.CompilerParams(use_tc_tiling_on_sc=True,
                                             needs_layout_passes=False),
    )
    def k(pack_hbm, prow_hbm, sub_hbm, out_hbm,
          prow_a, prow_b, sub_a, sub_b, rows_a, rows_b, h1t, sem_a, sem_b):
        wid = lax.axis_index("s") * nc + lax.axis_index("c")
        base = wid * lk_w
        iota = lax.iota(jnp.int32, 16)

        def load_and_fire(c, prow_v, sub_v, rows_v, sem):
            off = base + c * _C
            pltpu.sync_copy(prow_hbm.at[pl.ds(off, _C)], prow_v)
            pltpu.sync_copy(sub_hbm.at[pl.ds(off, _C)], sub_v)
            for k4 in range(0, _C, 128):
                sz = min(128, _C - k4)
                pltpu.make_async_copy(
                    pack_hbm.at[prow_v.at[pl.ds(k4, sz)]],
                    rows_v.at[pl.ds(k4, sz)],
                    sem,
                ).start()

        def wait_rows(prow_v, rows_v, sem):
            for k4 in range(0, _C, 128):
                sz = min(128, _C - k4)
                pltpu.make_async_copy(
                    pack_hbm.at[prow_v.at[pl.ds(k4, sz)]],
                    rows_v.at[pl.ds(k4, sz)],
                    sem,
                ).wait()

        def extract(c, sub_v, rows_v):
            coff = c * _S
            sub16 = sub_v[pl.ds(0, 16)]
            for e in range(_EMB):
                val = plsc.load_gather(rows_v, [iota, sub16 + e])
                h1t[e, pl.ds(coff, 16)] = val

            def fbody(f, carry):
                s16 = sub_v[pl.ds(f * _S, 16)]
                r16 = f * _S + iota
                for e in range(_EMB):
                    val = plsc.load_gather(rows_v, [r16, s16 + e])
                    plsc.addupdate(h1t.at[e, pl.ds(coff, 16)], val)
                return carry

            lax.fori_loop(1, _F, fbody, 0)

        load_and_fire(0, prow_a, sub_a, rows_a, sem_a)

        def two(cbase, carry):
            @pl.when(cbase + 1 < n_chunks)
            def _():
                load_and_fire(cbase + 1, prow_b, sub_b, rows_b, sem_b)
            wait_rows(prow_a, rows_a, sem_a)
            extract(cbase, sub_a, rows_a)

            @pl.when(cbase + 2 < n_chunks)
            def _():
                load_and_fire(cbase + 2, prow_a, sub_a, rows_a, sem_a)

            @pl.when(cbase + 1 < n_chunks)
            def _():
                wait_rows(prow_b, rows_b, sem_b)
                extract(cbase + 1, sub_b, rows_b)
            return carry

        lax.fori_loop(0, n_chunks // 2, lambda i, c: two(i * 2, c), 0)
        pltpu.sync_copy(h1t, out_hbm.at[wid])

    return k(packflat, prow, sub32)


def _mlp_body(h1t_ref, xn_ref, eye_ref, w1n_ref, b1_ref, w2_ref, b2_ref,
              w3_ref, b3_ref, out_ref):
    h1t = h1t_ref[0]
    h1 = lax.dot_general(h1t, eye_ref[...], (((0,), (0,)), ((), ())),
                         preferred_element_type=jnp.float32)
    h1 = h1 + jnp.dot(xn_ref[...], w1n_ref[...],
                      preferred_element_type=jnp.float32)
    h = jnp.maximum(h1 + b1_ref[...], 0.0)
    h = jnp.maximum(
        jnp.dot(h, w2_ref[...], preferred_element_type=jnp.float32)
        + b2_ref[...], 0.0)
    out_ref[...] = (jnp.dot(h, w3_ref[...], preferred_element_type=jnp.float32)
                    + b3_ref[...])


@jax.jit
def _tc_mlp(h1t_all, x_num, eye, w1n, b1, w2, b2, w3, b3):
    b = x_num.shape[0]
    nw = h1t_all.shape[0]
    blk = b // nw
    return pl.pallas_call(
        _mlp_body,
        grid=(nw,),
        in_specs=[
            pl.BlockSpec((1, _EMB, blk), lambda i: (i, 0, 0)),
            pl.BlockSpec((blk, _NN), lambda i: (i, 0)),
            pl.BlockSpec((_EMB, _EMB), lambda i: (0, 0)),
            pl.BlockSpec((_NN, 32), lambda i: (0, 0)),
            pl.BlockSpec((1, 32), lambda i: (0, 0)),
            pl.BlockSpec((32, 16), lambda i: (0, 0)),
            pl.BlockSpec((1, 16), lambda i: (0, 0)),
            pl.BlockSpec((16, 1), lambda i: (0, 0)),
            pl.BlockSpec((1, 1), lambda i: (0, 0)),
        ],
        out_specs=pl.BlockSpec((blk, 1), lambda i: (i, 0)),
        out_shape=jax.ShapeDtypeStruct((b, 1), jnp.float32),
        compiler_params=pltpu.CompilerParams(
            dimension_semantics=("arbitrary",)),
    )(h1t_all, x_num, eye, w1n, b1, w2, b2, w3, b3)


def kernel(X_num, X_cat, tables, W1, b1, W2, b2, W3, b3):
    b = X_num.shape[0]
    nw = 32
    per_w = b // nw
    n_chunks = per_w // _S

    tables_t = jnp.transpose(tables, (0, 2, 1))
    w1e = W1[_NN:].reshape(_F, _EMB, 32)
    pack = _tc_project(tables_t, w1e)
    packflat = pack.reshape(_F * _RPF, 128)

    v = X_cat
    prow = (jnp.arange(_F, dtype=jnp.int32)[None, :] * _RPF
            + (v // _VBLK) * 128 + (v % 128))
    sub32 = ((v // 128) % 4) * 32
    def perm(a):
        return (a.reshape(nw, n_chunks, _S, _F)
                 .transpose(0, 1, 3, 2)
                 .reshape(b * _F))
    h1t_all = _sc_gather_acc(packflat, perm(prow), perm(sub32), b)

    eye = jnp.eye(_EMB, dtype=jnp.float32)
    return _tc_mlp(h1t_all, X_num, eye, W1[:_NN], b1.reshape(1, 32),
                   W2, b2.reshape(1, 16), W3, b3.reshape(1, 1))


# speedup vs baseline: 3.8042x; 3.8042x over previous
import functools

import jax
import jax.numpy as jnp
from jax import lax
from jax.experimental import pallas as pl
from jax.experimental.pallas import tpu as pltpu
from jax.experimental.pallas import tpu_sc as plsc

_F = 26
_EMB = 32
_VOCAB = 100000
_NN = 13
_VBLK = 512
_NBLK = (_VOCAB + _VBLK - 1) // _VBLK
_RPF = _NBLK * (_VBLK // 4)
_S = 16
_C = _F * _S


def _proj_body(t_ref, w_ref, out_ref):
    for fi in range(_F):
        t = t_ref[fi].astype(jnp.bfloat16)
        w = w_ref[fi].astype(jnp.bfloat16)
        p = lax.dot_general(t, w, (((0,), (0,)), ((), ())),
                            preferred_element_type=jnp.float32)
        out_ref[fi, :, 0:32] = p[0:128]
        out_ref[fi, :, 32:64] = p[128:256]
        out_ref[fi, :, 64:96] = p[256:384]
        out_ref[fi, :, 96:128] = p[384:512]


@jax.jit
def _tc_project(tables_t, w1e):
    return pl.pallas_call(
        _proj_body,
        grid=(_NBLK,),
        in_specs=[
            pl.BlockSpec((_F, _EMB, _VBLK), lambda v: (0, 0, v)),
            pl.BlockSpec((_F, _EMB, 32), lambda v: (0, 0, 0)),
        ],
        out_specs=pl.BlockSpec((_F, 128, 128), lambda v: (0, v, 0)),
        out_shape=jax.ShapeDtypeStruct((_F, _RPF, 128), jnp.float32),
        compiler_params=pltpu.CompilerParams(
            dimension_semantics=("arbitrary",)),
    )(tables_t, w1e)


@functools.partial(jax.jit, static_argnums=(3,))
def _sc_gather_acc(packflat, prow, sub32, batch):
    info = plsc.get_sparse_core_info()
    nc = info.num_cores
    nw = nc * info.num_subcores
    per_w = batch // nw
    n_chunks = per_w // _S
    lk_w = per_w * _F

    mesh = plsc.VectorSubcoreMesh(core_axis_name="c", subcore_axis_name="s")

    @functools.partial(
        pl.kernel,
        mesh=mesh,
        out_type=jax.ShapeDtypeStruct((nw, _EMB, per_w), jnp.float32),
        scratch_types=[
            pltpu.VMEM((_C,), jnp.int32),
            pltpu.VMEM((_C,), jnp.int32),
            pltpu.VMEM((_C,), jnp.int32),
            pltpu.VMEM((_C,), jnp.int32),
            pltpu.VMEM((_C, 128), jnp.float32),
            pltpu.VMEM((_C, 128), jnp.float32),
            pltpu.VMEM((_EMB, per_w), jnp.float32),
            pltpu.SemaphoreType.DMA,
            pltpu.SemaphoreType.DMA,
        ],
        compiler_params=pltpu.CompilerParams(use_tc_tiling_on_sc=True,
                                             needs_layout_passes=False),
    )
    def k(pack_hbm, prow_hbm, sub_hbm, out_hbm,
          prow_a, prow_b, sub_a, sub_b, rows_a, rows_b, h1t, sem_a, sem_b):
        wid = lax.axis_index("s") * nc + lax.axis_index("c")
        base = wid * lk_w
        iota = lax.iota(jnp.int32, 16)

        def load_and_fire(c, prow_v, sub_v, rows_v, sem):
            off = base + c * _C
            pltpu.sync_copy(prow_hbm.at[pl.ds(off, _C)], prow_v)
            pltpu.sync_copy(sub_hbm.at[pl.ds(off, _C)], sub_v)
            for k4 in range(0, _C, 128):
                sz = min(128, _C - k4)
                pltpu.make_async_copy(
                    pack_hbm.at[prow_v.at[pl.ds(k4, sz)]],
                    rows_v.at[pl.ds(k4, sz)],
                    sem,
                ).start()

        def wait_rows(prow_v, rows_v, sem):
            for k4 in range(0, _C, 128):
                sz = min(128, _C - k4)
                pltpu.make_async_copy(
                    pack_hbm.at[prow_v.at[pl.ds(k4, sz)]],
                    rows_v.at[pl.ds(k4, sz)],
                    sem,
                ).wait()

        def extract(c, sub_v, rows_v):
            coff = c * _S
            sub16 = sub_v[pl.ds(0, 16)]
            for e in range(_EMB):
                val = plsc.load_gather(rows_v, [iota, sub16 + e])
                h1t[e, pl.ds(coff, 16)] = val

            def fbody(f, carry):
                s16 = sub_v[pl.ds(f * _S, 16)]
                r16 = f * _S + iota
                for e in range(_EMB):
                    val = plsc.load_gather(rows_v, [r16, s16 + e])
                    plsc.addupdate(h1t.at[e, pl.ds(coff, 16)], val)
                return carry

            lax.fori_loop(1, _F, fbody, 0)

        load_and_fire(0, prow_a, sub_a, rows_a, sem_a)

        def two(cbase, carry):
            @pl.when(cbase + 1 < n_chunks)
            def _():
                load_and_fire(cbase + 1, prow_b, sub_b, rows_b, sem_b)
            wait_rows(prow_a, rows_a, sem_a)
            extract(cbase, sub_a, rows_a)

            @pl.when(cbase + 2 < n_chunks)
            def _():
                load_and_fire(cbase + 2, prow_a, sub_a, rows_a, sem_a)

            @pl.when(cbase + 1 < n_chunks)
            def _():
                wait_rows(prow_b, rows_b, sem_b)
                extract(cbase + 1, sub_b, rows_b)
            return carry

        lax.fori_loop(0, n_chunks // 2, lambda i, c: two(i * 2, c), 0)
        pltpu.sync_copy(h1t, out_hbm.at[wid])

    return k(packflat, prow, sub32)


def _mlp_body(h1t_ref, xn_ref, eye_ref, w1n_ref, b1_ref, w2_ref, b2_ref,
              w3_ref, b3_ref, out_ref):
    h1t = h1t_ref[0]
    h1 = lax.dot_general(h1t, eye_ref[...], (((0,), (0,)), ((), ())),
                         preferred_element_type=jnp.float32)
    h1 = h1 + jnp.dot(xn_ref[...], w1n_ref[...],
                      preferred_element_type=jnp.float32)
    h = jnp.maximum(h1 + b1_ref[...], 0.0)
    h = jnp.maximum(
        jnp.dot(h, w2_ref[...], preferred_element_type=jnp.float32)
        + b2_ref[...], 0.0)
    out_ref[...] = (jnp.dot(h, w3_ref[...], preferred_element_type=jnp.float32)
                    + b3_ref[...])


@jax.jit
def _tc_mlp(h1t_all, x_num, eye, w1n, b1, w2, b2, w3, b3):
    b = x_num.shape[0]
    nw = h1t_all.shape[0]
    blk = b // nw
    return pl.pallas_call(
        _mlp_body,
        grid=(nw,),
        in_specs=[
            pl.BlockSpec((1, _EMB, blk), lambda i: (i, 0, 0)),
            pl.BlockSpec((blk, _NN), lambda i: (i, 0)),
            pl.BlockSpec((_EMB, _EMB), lambda i: (0, 0)),
            pl.BlockSpec((_NN, 32), lambda i: (0, 0)),
            pl.BlockSpec((1, 32), lambda i: (0, 0)),
            pl.BlockSpec((32, 16), lambda i: (0, 0)),
            pl.BlockSpec((1, 16), lambda i: (0, 0)),
            pl.BlockSpec((16, 1), lambda i: (0, 0)),
            pl.BlockSpec((1, 1), lambda i: (0, 0)),
        ],
        out_specs=pl.BlockSpec((blk, 1), lambda i: (i, 0)),
        out_shape=jax.ShapeDtypeStruct((b, 1), jnp.float32),
        compiler_params=pltpu.CompilerParams(
            dimension_semantics=("arbitrary",)),
    )(h1t_all, x_num, eye, w1n, b1, w2, b2, w3, b3)


def kernel(X_num, X_cat, tables, W1, b1, W2, b2, W3, b3):
    b = X_num.shape[0]
    nw = 32
    per_w = b // nw
    n_chunks = per_w // _S

    tables_t = jnp.transpose(tables, (0, 2, 1))
    w1e = W1[_NN:].reshape(_F, _EMB, 32)
    pack = _tc_project(tables_t, w1e)
    packflat = pack.reshape(_F * _RPF, 128)

    v = X_cat
    prow = (jnp.arange(_F, dtype=jnp.int32)[None, :] * _RPF
            + (v // _VBLK) * 128 + (v % 128))
    sub32 = ((v // 128) % 4) * 32
    def perm(a):
        return (a.reshape(nw, n_chunks, _S, _F)
                 .transpose(0, 1, 3, 2)
                 .reshape(b * _F))
    h1t_all = _sc_gather_acc(packflat, perm(prow), perm(sub32), b)

    eye = jnp.eye(_EMB, dtype=jnp.float32)
    return _tc_mlp(h1t_all, X_num, eye, W1[:_NN], b1.reshape(1, 32),
                   W2, b2.reshape(1, 16), W3, b3.reshape(1, 1))


# speedup vs baseline: 4.0286x; 1.0590x over previous
import functools

import jax
import jax.numpy as jnp
from jax import lax
from jax.experimental import pallas as pl
from jax.experimental.pallas import tpu as pltpu
from jax.experimental.pallas import tpu_sc as plsc

_F = 26
_G = 2
_FG = _F // _G
_EMB = 32
_VOCAB = 100000
_NN = 13
_VBLK = 512
_NBLK = (_VOCAB + _VBLK - 1) // _VBLK
_RPF = _NBLK * 128
_S = 16


def _proj_body(t_ref, w_ref, out_ref):
    for fi in range(_FG):
        t = t_ref[fi].astype(jnp.bfloat16)
        w = w_ref[fi].astype(jnp.bfloat16)
        p = lax.dot_general(t, w, (((0,), (0,)), ((), ())),
                            preferred_element_type=jnp.float32)
        out_ref[fi, :, 0:32] = p[0:128]
        out_ref[fi, :, 32:64] = p[128:256]
        out_ref[fi, :, 64:96] = p[256:384]
        out_ref[fi, :, 96:128] = p[384:512]


@functools.partial(jax.jit, static_argnums=(2,))
def _tc_project(tables_t, w1e, g):
    return pl.pallas_call(
        _proj_body,
        grid=(_NBLK,),
        in_specs=[
            pl.BlockSpec((_FG, _EMB, _VBLK), lambda v: (g, 0, v)),
            pl.BlockSpec((_FG, _EMB, 32), lambda v: (g, 0, 0)),
        ],
        out_specs=pl.BlockSpec((_FG, 128, 128), lambda v: (0, v, 0)),
        out_shape=jax.ShapeDtypeStruct((_FG, _RPF, 128), jnp.float32),
        compiler_params=pltpu.CompilerParams(
            dimension_semantics=("arbitrary",)),
    )(tables_t, w1e)


@functools.partial(jax.jit, static_argnums=(3,))
def _sc_gather_acc(packflat, prow, sub32, batch):
    nlk = _FG * _S
    info = plsc.get_sparse_core_info()
    nc = info.num_cores
    nw = nc * info.num_subcores
    per_w = batch // nw
    n_chunks = per_w // _S
    lk_w = per_w * _FG

    mesh = plsc.VectorSubcoreMesh(core_axis_name="c", subcore_axis_name="s")

    @functools.partial(
        pl.kernel,
        mesh=mesh,
        out_type=jax.ShapeDtypeStruct((nw, _EMB, per_w), jnp.float32),
        scratch_types=[
            pltpu.VMEM((nlk,), jnp.int32),
            pltpu.VMEM((nlk,), jnp.int32),
            pltpu.VMEM((nlk,), jnp.int32),
            pltpu.VMEM((nlk,), jnp.int32),
            pltpu.VMEM((nlk, 128), jnp.float32),
            pltpu.VMEM((nlk, 128), jnp.float32),
            pltpu.VMEM((_EMB, per_w), jnp.float32),
            pltpu.SemaphoreType.DMA,
            pltpu.SemaphoreType.DMA,
        ],
        compiler_params=pltpu.CompilerParams(use_tc_tiling_on_sc=True,
                                             needs_layout_passes=False),
    )
    def k(pack_hbm, prow_hbm, sub_hbm, out_hbm,
          prow_a, prow_b, sub_a, sub_b, rows_a, rows_b, h1t, sem_a, sem_b):
        wid = lax.axis_index("s") * nc + lax.axis_index("c")
        base = wid * lk_w
        iota = lax.iota(jnp.int32, 16)

        def load_and_fire(c, prow_v, sub_v, rows_v, sem):
            off = base + c * nlk
            pltpu.sync_copy(prow_hbm.at[pl.ds(off, nlk)], prow_v)
            pltpu.sync_copy(sub_hbm.at[pl.ds(off, nlk)], sub_v)
            for k4 in range(0, nlk, 128):
                sz = min(128, nlk - k4)
                pltpu.make_async_copy(
                    pack_hbm.at[prow_v.at[pl.ds(k4, sz)]],
                    rows_v.at[pl.ds(k4, sz)],
                    sem,
                ).start()

        def wait_rows(prow_v, rows_v, sem):
            for k4 in range(0, nlk, 128):
                sz = min(128, nlk - k4)
                pltpu.make_async_copy(
                    pack_hbm.at[prow_v.at[pl.ds(k4, sz)]],
                    rows_v.at[pl.ds(k4, sz)],
                    sem,
                ).wait()

        def extract(c, sub_v, rows_v):
            coff = c * _S
            sub16 = sub_v[pl.ds(0, 16)]
            for e in range(_EMB):
                val = plsc.load_gather(rows_v, [iota, sub16 + e])
                h1t[e, pl.ds(coff, 16)] = val

            def fbody(f, carry):
                s16 = sub_v[pl.ds(f * _S, 16)]
                r16 = f * _S + iota
                for e in range(_EMB):
                    val = plsc.load_gather(rows_v, [r16, s16 + e])
                    plsc.addupdate(h1t.at[e, pl.ds(coff, 16)], val)
                return carry

            lax.fori_loop(1, _FG, fbody, 0)

        load_and_fire(0, prow_a, sub_a, rows_a, sem_a)

        def two(cbase, carry):
            @pl.when(cbase + 1 < n_chunks)
            def _():
                load_and_fire(cbase + 1, prow_b, sub_b, rows_b, sem_b)
            wait_rows(prow_a, rows_a, sem_a)
            extract(cbase, sub_a, rows_a)

            @pl.when(cbase + 2 < n_chunks)
            def _():
                load_and_fire(cbase + 2, prow_a, sub_a, rows_a, sem_a)

            @pl.when(cbase + 1 < n_chunks)
            def _():
                wait_rows(prow_b, rows_b, sem_b)
                extract(cbase + 1, sub_b, rows_b)
            return carry

        lax.fori_loop(0, n_chunks // 2, lambda i, c: two(i * 2, c), 0)
        pltpu.sync_copy(h1t, out_hbm.at[wid])

    return k(packflat, prow, sub32)


def _mlp_body(h0_ref, h1_ref, xn_ref, eye_ref, w1n_ref, b1_ref, w2_ref,
              b2_ref, w3_ref, b3_ref, out_ref):
    h1t = h0_ref[0] + h1_ref[0]
    h1 = lax.dot_general(h1t, eye_ref[...], (((0,), (0,)), ((), ())),
                         preferred_element_type=jnp.float32)
    h1 = h1 + jnp.dot(xn_ref[...], w1n_ref[...],
                      preferred_element_type=jnp.float32)
    h = jnp.maximum(h1 + b1_ref[...], 0.0)
    h = jnp.maximum(
        jnp.dot(h, w2_ref[...], preferred_element_type=jnp.float32)
        + b2_ref[...], 0.0)
    out_ref[...] = (jnp.dot(h, w3_ref[...], preferred_element_type=jnp.float32)
                    + b3_ref[...])


@jax.jit
def _tc_mlp(h0, h1, x_num, eye, w1n, b1, w2, b2, w3, b3):
    b = x_num.shape[0]
    nw = h0.shape[0]
    blk = b // nw
    return pl.pallas_call(
        _mlp_body,
        grid=(nw,),
        in_specs=[
            pl.BlockSpec((1, _EMB, blk), lambda i: (i, 0, 0)),
            pl.BlockSpec((1, _EMB, blk), lambda i: (i, 0, 0)),
            pl.BlockSpec((blk, _NN), lambda i: (i, 0)),
            pl.BlockSpec((_EMB, _EMB), lambda i: (0, 0)),
            pl.BlockSpec((_NN, 32), lambda i: (0, 0)),
            pl.BlockSpec((1, 32), lambda i: (0, 0)),
            pl.BlockSpec((32, 16), lambda i: (0, 0)),
            pl.BlockSpec((1, 16), lambda i: (0, 0)),
            pl.BlockSpec((16, 1), lambda i: (0, 0)),
            pl.BlockSpec((1, 1), lambda i: (0, 0)),
        ],
        out_specs=pl.BlockSpec((blk, 1), lambda i: (i, 0)),
        out_shape=jax.ShapeDtypeStruct((b, 1), jnp.float32),
        compiler_params=pltpu.CompilerParams(
            dimension_semantics=("arbitrary",)),
    )(h0, h1, x_num, eye, w1n, b1, w2, b2, w3, b3)


def kernel(X_num, X_cat, tables, W1, b1, W2, b2, W3, b3):
    b = X_num.shape[0]
    nw = 32
    per_w = b // nw
    n_chunks = per_w // _S

    tables_t = jnp.transpose(tables, (0, 2, 1))
    w1e = W1[_NN:].reshape(_F, _EMB, 32)

    v = X_cat
    prow = (jnp.arange(_FG, dtype=jnp.int32)[None, :] * _RPF
            + (v.reshape(b, _G, _FG) // _VBLK) * 128
            + (v.reshape(b, _G, _FG) % 128))
    sub32 = ((v.reshape(b, _G, _FG) // 128) % 4) * 32

    def perm(a, g):
        return (a[:, g].reshape(nw, n_chunks, _S, _FG)
                 .transpose(0, 1, 3, 2)
                 .reshape(b * _FG))

    hs = []
    for g in range(_G):
        pack = _tc_project(tables_t, w1e, g)
        packflat = pack.reshape(_FG * _RPF, 128)
        hs.append(_sc_gather_acc(packflat, perm(prow, g), perm(sub32, g), b))

    eye = jnp.eye(_EMB, dtype=jnp.float32)
    return _tc_mlp(hs[0], hs[1], X_num, eye, W1[:_NN], b1.reshape(1, 32),
                   W2, b2.reshape(1, 16), W3, b3.reshape(1, 1))
